# baseline (speedup 1.0000x reference)
import functools
import math

import jax
import jax.numpy as jnp
from jax import lax
from jax.experimental import pallas as pl
from jax.experimental.pallas import tpu as pltpu
from jax.experimental.pallas import tpu_sc as plsc

_LANES = 16


@functools.partial(jax.jit, static_argnums=(3, 4))
def _embed_lookup_t(inputs, tok_t, pos_t, n_batch, scale):
    d, v = tok_t.shape
    b, s = inputs.shape
    n = b * s
    cut = -(-(v // 2) // 128) * 128
    lens = [cut, v - cut]
    offs = [0, cut]
    info = plsc.get_sparse_core_info()
    n_workers = info.num_cores * info.num_subcores
    dims_per_w = d // n_workers
    mesh = plsc.VectorSubcoreMesh(core_axis_name="c", subcore_axis_name="s")

    @functools.partial(
        pl.kernel,
        mesh=mesh,
        compiler_params=pltpu.CompilerParams(needs_layout_passes=False),
        out_type=jax.ShapeDtypeStruct((b, d, s), jnp.float32),
        scratch_types=[
            pltpu.VMEM((n,), jnp.int32),
            pltpu.VMEM((lens[0],), jnp.float32),
            pltpu.VMEM((lens[1],), jnp.float32),
            pltpu.VMEM((n,), jnp.float32),
            pltpu.VMEM((n,), jnp.float32),
            pltpu.SemaphoreType.DMA,
            pltpu.SemaphoreType.DMA,
            pltpu.SemaphoreType.DMA,
            pltpu.SemaphoreType.DMA,
            pltpu.SemaphoreType.DMA,
        ],
    )
    def run(idx_hbm, tok_hbm, pos_hbm, out_hbm,
            idx_v, row_a, row_b, out_0, out_1, isem, asem, bsem, psem, wsem):
        wid = lax.axis_index("s") * info.num_cores + lax.axis_index("c")
        dim0 = wid * dims_per_w
        rows = [row_a, row_b]
        sems = [asem, bsem]
        outs = [out_0, out_1]
        iota = lax.iota(jnp.int32, _LANES)

        idx_copies = [
            pltpu.async_copy(idx_hbm.at[bb], idx_v.at[pl.ds(bb * s, s)], isem)
            for bb in range(b)
        ]
        prefills = [
            pltpu.async_copy(pos_hbm.at[dim0 + j, pl.ds(0, s)],
                             outs[j % 2].at[pl.ds(bb * s, s)], psem)
            for j in range(dims_per_w) for bb in range(b)
        ]
        half_copies = [
            pltpu.async_copy(tok_hbm.at[dim0, pl.ds(offs[h], lens[h])],
                             rows[h], sems[h])
            for h in range(2)
        ]
        out_copies = []

        n_chunks = n // _LANES
        n_streams = 4
        span = n_chunks // n_streams

        def make_pass(row_ref, out_ref, h):
            def body(k, _):
                gs = []
                for t in range(n_streams):
                    kk = k + t * span
                    iv = idx_v[pl.ds(kk * _LANES, _LANES)]
                    if h == 0:
                        g = plsc.load_gather(
                            row_ref, [jnp.minimum(iv, cut - 1)])
                        m = iv < cut
                    else:
                        g = plsc.load_gather(
                            row_ref, [jnp.maximum(iv, cut) - cut])
                        m = iv >= cut
                    gs.append((kk * _LANES + iota, g, m))
                for pos_ix, g, m in gs:
                    plsc.addupdate_scatter(out_ref, [pos_ix],
                                           g * scale, mask=m)
                return ()
            return body

        for c in idx_copies:
            c.wait()
        for c in prefills:
            c.wait()
        for j in range(dims_per_w):
            out_v = outs[j % 2]
            for h in range(2):
                half_copies[j * 2 + h].wait()
                lax.fori_loop(0, span, make_pass(rows[h], out_v, h), (),
                              unroll=4)
                if j + 1 < dims_per_w:
                    half_copies.append(pltpu.async_copy(
                        tok_hbm.at[dim0 + j + 1, pl.ds(offs[h], lens[h])],
                        rows[h], sems[h]))
            for bb in range(b):
                out_copies.append(pltpu.async_copy(
                    out_v.at[pl.ds(bb * s, s)],
                    out_hbm.at[bb, dim0 + j, pl.ds(0, s)], wsem))
        for c in out_copies:
            c.wait()

    return run(inputs, tok_t, pos_t)


def kernel(inputs, token_table, pos_table):
    b, s = inputs.shape
    d = token_table.shape[1]
    scale = float(math.sqrt(d))
    out_t = _embed_lookup_t(inputs.astype(jnp.int32), token_table.T,
                            pos_table.T, b, scale)
    return out_t.transpose(0, 2, 1)


# speedup vs baseline: 1.0782x; 1.0782x over previous
import functools
import math

import jax
import jax.numpy as jnp
from jax import lax
from jax.experimental import pallas as pl
from jax.experimental.pallas import tpu as pltpu
from jax.experimental.pallas import tpu_sc as plsc

_LANES = 16


@functools.partial(jax.jit, static_argnums=(3, 4))
def _embed_lookup_t(inputs, tok_t, pos_t, n_batch, scale):
    d, v = tok_t.shape
    b, s = inputs.shape
    n = b * s
    info = plsc.get_sparse_core_info()
    n_workers = info.num_cores * info.num_subcores
    dims_per_w = d // n_workers
    mesh = plsc.VectorSubcoreMesh(core_axis_name="c", subcore_axis_name="s")

    @functools.partial(
        pl.kernel,
        mesh=mesh,
        compiler_params=pltpu.CompilerParams(needs_layout_passes=False),
        out_type=jax.ShapeDtypeStruct((b, d, s), jnp.float32),
        scratch_types=[
            pltpu.VMEM((n,), jnp.int32),
            pltpu.VMEM((v,), jnp.float32),
            pltpu.VMEM((n,), jnp.float32),
            pltpu.VMEM((n,), jnp.float32),
            pltpu.SemaphoreType.DMA,
            pltpu.SemaphoreType.DMA,
            pltpu.SemaphoreType.DMA,
            pltpu.SemaphoreType.DMA,
        ],
    )
    def run(idx_hbm, tok_hbm, pos_hbm, out_hbm,
            idx_v, row_v, out_0, out_1, isem, rsem, psem, wsem):
        wid = lax.axis_index("s") * info.num_cores + lax.axis_index("c")
        dim0 = wid * dims_per_w
        outs = [out_0, out_1]

        idx_copies = [
            pltpu.async_copy(idx_hbm.at[bb], idx_v.at[pl.ds(bb * s, s)], isem)
            for bb in range(b)
        ]
        prefills = [
            pltpu.async_copy(pos_hbm.at[dim0, pl.ds(0, s)],
                             outs[0].at[pl.ds(bb * s, s)], psem)
            for bb in range(b)
        ]
        row_copy = pltpu.async_copy(tok_hbm.at[dim0], row_v, rsem)
        out_copies = []

        for c in idx_copies:
            c.wait()
        for j in range(dims_per_w):
            dim = dim0 + j
            out_v = outs[j % 2]
            for c in prefills:
                c.wait()
            row_copy.wait()

            n_chunks = n // _LANES
            n_streams = 8
            span = n_chunks // n_streams

            def body(k, _):
                gs = []
                for t in range(n_streams):
                    sl = pl.ds((k + t * span) * _LANES, _LANES)
                    gs.append((sl, plsc.load_gather(row_v, [idx_v[sl]])))
                for sl, g in gs:
                    plsc.addupdate(out_v.at[sl], g * scale)
                return ()

            lax.fori_loop(0, span, body, (), unroll=2)

            if j + 1 < dims_per_w:
                row_copy = pltpu.async_copy(tok_hbm.at[dim + 1], row_v, rsem)
                prefills = [
                    pltpu.async_copy(pos_hbm.at[dim + 1, pl.ds(0, s)],
                                     outs[(j + 1) % 2].at[pl.ds(bb * s, s)],
                                     psem)
                    for bb in range(b)
                ]
            for bb in range(b):
                out_copies.append(pltpu.async_copy(
                    out_v.at[pl.ds(bb * s, s)],
                    out_hbm.at[bb, dim, pl.ds(0, s)], wsem))
        for c in out_copies:
            c.wait()

    return run(inputs, tok_t, pos_t)


def kernel(inputs, token_table, pos_table):
    b, s = inputs.shape
    d = token_table.shape[1]
    scale = float(math.sqrt(d))
    out_t = _embed_lookup_t(inputs.astype(jnp.int32), token_table.T,
                            pos_table.T, b, scale)
    return out_t.transpose(0, 2, 1)


# speedup vs baseline: 1.0802x; 1.0019x over previous
import functools
import math

import jax
import jax.numpy as jnp
from jax import lax
from jax.experimental import pallas as pl
from jax.experimental.pallas import tpu as pltpu
from jax.experimental.pallas import tpu_sc as plsc

_LANES = 16


@functools.partial(jax.jit, static_argnums=(3, 4))
def _embed_lookup_t(inputs, tok_t, pos_t, n_batch, scale):
    d, v = tok_t.shape
    b, s = inputs.shape
    n = b * s
    info = plsc.get_sparse_core_info()
    n_workers = info.num_cores * info.num_subcores
    dims_per_w = d // n_workers
    mesh = plsc.VectorSubcoreMesh(core_axis_name="c", subcore_axis_name="s")

    @functools.partial(
        pl.kernel,
        mesh=mesh,
        compiler_params=pltpu.CompilerParams(needs_layout_passes=False, skip_device_barrier=True, disable_bounds_checks=True),
        out_type=jax.ShapeDtypeStruct((b, d, s), jnp.float32),
        scratch_types=[
            pltpu.VMEM((n,), jnp.int32),
            pltpu.VMEM((v,), jnp.float32),
            pltpu.VMEM((n,), jnp.float32),
            pltpu.VMEM((n,), jnp.float32),
            pltpu.SemaphoreType.DMA,
            pltpu.SemaphoreType.DMA,
            pltpu.SemaphoreType.DMA,
            pltpu.SemaphoreType.DMA,
        ],
    )
    def run(idx_hbm, tok_hbm, pos_hbm, out_hbm,
            idx_v, row_v, out_0, out_1, isem, rsem, psem, wsem):
        wid = lax.axis_index("s") * info.num_cores + lax.axis_index("c")
        dim0 = wid * dims_per_w
        outs = [out_0, out_1]

        idx_copies = [
            pltpu.async_copy(idx_hbm.at[bb], idx_v.at[pl.ds(bb * s, s)], isem)
            for bb in range(b)
        ]
        prefills = [
            pltpu.async_copy(pos_hbm.at[dim0, pl.ds(0, s)],
                             outs[0].at[pl.ds(bb * s, s)], psem)
            for bb in range(b)
        ]
        row_copy = pltpu.async_copy(tok_hbm.at[dim0], row_v, rsem)
        out_copies = []

        for c in idx_copies:
            c.wait()
        for j in range(dims_per_w):
            dim = dim0 + j
            out_v = outs[j % 2]
            for c in prefills:
                c.wait()
            row_copy.wait()

            n_chunks = n // _LANES
            n_streams = 8
            span = n_chunks // n_streams

            def body(k, _):
                gs = []
                for t in range(n_streams):
                    sl = pl.ds((k + t * span) * _LANES, _LANES)
                    gs.append((sl, plsc.load_gather(row_v, [idx_v[sl]])))
                for sl, g in gs:
                    plsc.addupdate(out_v.at[sl], g * scale)
                return ()

            lax.fori_loop(0, span, body, (), unroll=2)

            if j + 1 < dims_per_w:
                row_copy = pltpu.async_copy(tok_hbm.at[dim + 1], row_v, rsem)
                prefills = [
                    pltpu.async_copy(pos_hbm.at[dim + 1, pl.ds(0, s)],
                                     outs[(j + 1) % 2].at[pl.ds(bb * s, s)],
                                     psem)
                    for bb in range(b)
                ]
            for bb in range(b):
                out_copies.append(pltpu.async_copy(
                    out_v.at[pl.ds(bb * s, s)],
                    out_hbm.at[bb, dim, pl.ds(0, s)], wsem))
        for c in out_copies:
            c.wait()

    return run(inputs, tok_t, pos_t)


def kernel(inputs, token_table, pos_table):
    b, s = inputs.shape
    d = token_table.shape[1]
    scale = float(math.sqrt(d))
    out_t = _embed_lookup_t(inputs.astype(jnp.int32), token_table.T,
                            pos_table.T, b, scale)
    return out_t.transpose(0, 2, 1)


# speedup vs baseline: 1.0814x; 1.0012x over previous
import functools
import math

import jax
import jax.numpy as jnp
from jax import lax
from jax.experimental import pallas as pl
from jax.experimental.pallas import tpu as pltpu
from jax.experimental.pallas import tpu_sc as plsc

_LANES = 16


@functools.partial(jax.jit, static_argnums=(3, 4))
def _embed_lookup_t(inputs, tok_t, pos_t, n_batch, scale):
    d, v = tok_t.shape
    b, s = inputs.shape
    n = b * s
    info = plsc.get_sparse_core_info()
    n_workers = info.num_cores * info.num_subcores
    dims_per_w = d // n_workers
    mesh = plsc.VectorSubcoreMesh(core_axis_name="c", subcore_axis_name="s")

    @functools.partial(
        pl.kernel,
        mesh=mesh,
        compiler_params=pltpu.CompilerParams(needs_layout_passes=False),
        out_type=jax.ShapeDtypeStruct((b, d, s), jnp.float32),
        scratch_types=[
            pltpu.VMEM((n,), jnp.int32),
            pltpu.VMEM((v,), jnp.float32),
            pltpu.VMEM((n,), jnp.float32),
            pltpu.VMEM((n,), jnp.float32),
            pltpu.SemaphoreType.DMA,
            pltpu.SemaphoreType.DMA,
            pltpu.SemaphoreType.DMA,
            pltpu.SemaphoreType.DMA,
        ],
    )
    def run(idx_hbm, tok_hbm, pos_hbm, out_hbm,
            idx_v, row_v, out_0, out_1, isem, rsem, psem, wsem):
        wid = lax.axis_index("s") * info.num_cores + lax.axis_index("c")
        dim0 = wid * dims_per_w
        outs = [out_0, out_1]

        idx_copies = [
            pltpu.async_copy(idx_hbm.at[bb], idx_v.at[pl.ds(bb * s, s)], isem)
            for bb in range(b)
        ]
        prefills = [
            pltpu.async_copy(pos_hbm.at[dim0, pl.ds(0, s)],
                             outs[0].at[pl.ds(bb * s, s)], psem)
            for bb in range(b)
        ]
        row_copy = pltpu.async_copy(tok_hbm.at[dim0], row_v, rsem)
        out_copies = []

        for c in idx_copies:
            c.wait()
        for j in range(dims_per_w):
            dim = dim0 + j
            out_v = outs[j % 2]
            for c in prefills:
                c.wait()
            row_copy.wait()

            n_chunks = n // _LANES
            n_streams = 8
            span = n_chunks // n_streams

            def body(k, _):
                gs = []
                for t in range(n_streams):
                    sl = pl.ds((k + t * span) * _LANES, _LANES)
                    gs.append((sl, plsc.load_gather(row_v, [idx_v[sl]])))
                for sl, g in gs:
                    plsc.addupdate(out_v.at[sl], g * scale)
                return ()

            lax.fori_loop(0, span, body, (), unroll=2)

            if j + 1 < dims_per_w:
                row_copy = pltpu.async_copy(tok_hbm.at[dim + 1], row_v, rsem)
                prefills = [
                    pltpu.async_copy(pos_hbm.at[dim + 1, pl.ds(0, s)],
                                     outs[(j + 1) % 2].at[pl.ds(bb * s, s)],
                                     psem)
                    for bb in range(b)
                ]
            for bb in range(b):
                out_copies.append(pltpu.async_copy(
                    out_v.at[pl.ds(bb * s, s)],
                    out_hbm.at[bb, dim, pl.ds(0, s)], wsem))
        for c in out_copies:
            c.wait()

    return run(inputs, tok_t, pos_t)


def kernel(inputs, token_table, pos_table):
    b, s = inputs.shape
    d = token_table.shape[1]
    scale = float(math.sqrt(d))
    out_t = _embed_lookup_t(inputs.astype(jnp.int32), token_table.T,
                            pos_table.T, b, scale)
    return out_t.transpose(0, 2, 1)
